# baseline (speedup 1.0000x reference)
import jax
import jax.numpy as jnp
from jax import lax
from jax.experimental import pallas as pl
from jax.experimental.pallas import tpu as pltpu
from jax.experimental.pallas import tpu_sc as plsc

_N = 1_000_000
_D = 64
_NC = 2
_NS = 16
_NW = _NC * _NS
_STEP = 400
_NSTEP = _N // _STEP
_MAXJ = -(-_NSTEP // _NW)


def _sc_body(x_hbm, o_hbm, buf0, buf1, sem_g0, sem_g1, sem_s0, sem_s1):
    wid = lax.axis_index("s") * _NC + lax.axis_index("c")
    bufs = (buf0, buf1)
    gsems = (sem_g0, sem_g1)
    ssems = (sem_s0, sem_s1)

    def valid(j):
        return wid + j * _NW < _NSTEP

    def start_of(j):
        idx = jnp.minimum(wid + j * _NW, _NSTEP - 1)
        return idx * _STEP

    def gather(j):
        b = j % 2
        return pltpu.make_async_copy(
            x_hbm.at[pl.ds(start_of(j), _STEP), :], bufs[b], gsems[b])

    def scatter(j):
        b = j % 2
        return pltpu.make_async_copy(
            bufs[b], o_hbm.at[pl.ds(start_of(j), _STEP), :], ssems[b])

    def guarded(j, fn):
        if j < _MAXJ - 1:
            fn()
        else:
            pl.when(valid(j))(fn)

    guarded(0, gather(0).start)
    for j in range(_MAXJ):
        b = j % 2
        if j + 1 < _MAXJ:
            if j >= 1:
                guarded(j - 1, scatter(j - 1).wait)
            guarded(j + 1, gather(j + 1).start)
        guarded(j, gather(j).wait)
        if j == 0:
            @pl.when(wid == 0)
            def _():
                two = jnp.full((16,), 2.0, jnp.float32)
                for c in range(_D // 16):
                    bufs[0][1, pl.ds(c * 16, 16)] = two
                    bufs[0][3, pl.ds(c * 16, 16)] = two
        guarded(j, scatter(j).start)
    guarded(_MAXJ - 2, scatter(_MAXJ - 2).wait)
    guarded(_MAXJ - 1, scatter(_MAXJ - 1).wait)


def kernel(x):
    f = pl.kernel(
        _sc_body,
        out_type=jax.ShapeDtypeStruct((_N, _D), jnp.float32),
        mesh=plsc.VectorSubcoreMesh(core_axis_name="c", subcore_axis_name="s"),
        scratch_types=[
            pltpu.VMEM((_STEP, _D), jnp.float32),
            pltpu.VMEM((_STEP, _D), jnp.float32),
            pltpu.SemaphoreType.DMA,
            pltpu.SemaphoreType.DMA,
            pltpu.SemaphoreType.DMA,
            pltpu.SemaphoreType.DMA,
        ],
    )
    return f(x)
